# Initial kernel scaffold; baseline (speedup 1.0000x reference)
import jax
import jax.numpy as jnp
from jax.experimental import pallas as pl


def kernel(user_emb, item_emb, adj_values, adj_indices):
    raise NotImplementedError("write your pallas kernel here")


# speedup vs baseline: 1.4132x; 1.4132x over previous
import functools

import jax
import jax.numpy as jnp
from jax import lax
from jax.experimental import pallas as pl
from jax.experimental.pallas import tpu as pltpu
from jax.experimental.pallas import tpu_sc as plsc

N_NODES = 100000
EMB = 32
E = 1600000
NT = 32
PT = E // NT
NB = 32
RANGE = 3200
RSZ = PT + NB * 16
EPAD = NT * RSZ + 528
DUMP = EPAD - 16
CH1 = 2000
KB2 = 512
ACCW = RANGE * EMB

_mesh = functools.partial(
    plsc.VectorSubcoreMesh, core_axis_name="c", subcore_axis_name="s")


def _wid():
  return lax.axis_index("s") * 2 + lax.axis_index("c")


def _bucket(d):
  q = lax.shift_right_logical(d, 7)
  return lax.shift_right_logical(q * 5243, 17)


def _io():
  return lax.iota(jnp.int32, 16)


def _ranks(sb, sbuf):
  io = _io()
  sbuf[...] = sb
  prev = plsc.load_gather(sbuf, [jnp.maximum(io - 1, 0)])
  nxt = plsc.load_gather(sbuf, [jnp.minimum(io + 1, 15)])
  first = jnp.logical_or(io == 0, sb != prev)
  is_end = jnp.logical_or(io == 15, sb != nxt)
  start = plsc.cummax(jnp.where(first, io, 0))
  rank = io - start
  return rank, is_end


def _partition_body(dst, src, val, srcs_o, dofs_o, vals_o, tbl_o,
                    dstb, srcb, valb, poso, srco, dofo, valo,
                    hist, runpos, startsv, tblv, sbuf, padidx, zbi, zbf):
  wid = _wid()
  slab = wid * PT
  regbase = wid * RSZ
  io = _io()

  hist[pl.ds(0, 16)] = jnp.zeros((16,), jnp.int32)
  hist[pl.ds(16, 16)] = jnp.zeros((16,), jnp.int32)

  def p1_chunk(c, _):
    pltpu.sync_copy(dst.at[pl.ds(slab + c * CH1, CH1)], dstb)
    def p1_vreg(i, _):
      d = dstb[pl.ds(i * 16, 16)]
      b = _bucket(d)
      sb, _ = plsc.sort_key_val(b, io)
      rank, is_end = _ranks(sb, sbuf)
      h = plsc.load_gather(hist, [sb])
      plsc.store_scatter(hist, [sb], h + rank + 1, mask=is_end)
      return 0
    lax.fori_loop(0, CH1 // 16, p1_vreg, 0)
    return 0
  lax.fori_loop(0, PT // CH1, p1_chunk, 0)

  h0 = hist[pl.ds(0, 16)]
  h1 = hist[pl.ds(16, 16)]
  p0 = jnp.bitwise_and(h0 + 15, -16)
  p1 = jnp.bitwise_and(h1 + 15, -16)
  c0 = plsc.cumsum(p0)
  c1 = plsc.cumsum(p1)
  tot0 = jnp.max(c0)
  s0 = regbase + (c0 - p0)
  s1 = regbase + (c1 - p1) + tot0
  startsv[pl.ds(0, 16)] = s0
  startsv[pl.ds(16, 16)] = s1
  runpos[pl.ds(0, 16)] = s0
  runpos[pl.ds(16, 16)] = s1

  def p2_chunk(c, _):
    coff = slab + c * CH1
    pltpu.sync_copy(dst.at[pl.ds(coff, CH1)], dstb)
    pltpu.sync_copy(src.at[pl.ds(coff, CH1)], srcb)
    pltpu.sync_copy(val.at[pl.ds(coff, CH1)], valb)
    def p2_vreg(i, _):
      d = dstb[pl.ds(i * 16, 16)]
      b = _bucket(d)
      sb, lanes = plsc.sort_key_val(b, i * 16 + io)
      rank, is_end = _ranks(sb, sbuf)
      base = plsc.load_gather(runpos, [sb])
      pos = base + rank
      plsc.store_scatter(runpos, [sb], pos + 1, mask=is_end)
      d_s = plsc.load_gather(dstb, [lanes])
      s_s = plsc.load_gather(srcb, [lanes])
      v_s = plsc.load_gather(valb, [lanes])
      doff_s = lax.shift_left(d_s - sb * RANGE, 5)
      poso[pl.ds(i * 16, 16)] = pos
      srco[pl.ds(i * 16, 16)] = s_s
      dofo[pl.ds(i * 16, 16)] = doff_s
      valo[pl.ds(i * 16, 16)] = v_s
      return 0
    lax.fori_loop(0, CH1 // 16, p2_vreg, 0)
    pltpu.sync_copy(srco, srcs_o.at[poso])
    pltpu.sync_copy(dofo, dofs_o.at[poso])
    pltpu.sync_copy(valo, vals_o.at[poso])
    return 0
  lax.fori_loop(0, PT // CH1, p2_chunk, 0)

  def zb_body(i, _):
    zbi[pl.ds(i * 16, 16)] = jnp.zeros((16,), jnp.int32)
    zbf[pl.ds(i * 16, 16)] = jnp.zeros((16,), jnp.float32)
    return 0
  lax.fori_loop(0, 32, zb_body, 0)

  rp0 = runpos[pl.ds(0, 16)]
  rp1 = runpos[pl.ds(16, 16)]
  pn0 = p0 - h0
  pn1 = p1 - h1
  for j in range(15):
    padidx[pl.ds(j * 32, 16)] = jnp.where(j < pn0, rp0 + j, DUMP)
    padidx[pl.ds(j * 32 + 16, 16)] = jnp.where(j < pn1, rp1 + j, DUMP)
  padidx[pl.ds(480, 16)] = jnp.full((16,), DUMP, jnp.int32)
  padidx[pl.ds(496, 16)] = jnp.full((16,), DUMP, jnp.int32)
  pltpu.sync_copy(zbi, srcs_o.at[padidx])
  pltpu.sync_copy(zbi, dofs_o.at[padidx])
  pltpu.sync_copy(zbf, vals_o.at[padidx])

  regend = regbase + jnp.max(c1) + tot0
  cap = jnp.where(wid < NT - 1, regbase + RSZ, DUMP)
  for blk in range(2):
    def tz_body(i, _):
      p = regend + blk * 512 + i * 16 + io
      padidx[pl.ds(i * 16, 16)] = jnp.where(p < cap, p, DUMP)
      return 0
    lax.fori_loop(0, 32, tz_body, 0)
    pltpu.sync_copy(zbi, srcs_o.at[padidx])
    pltpu.sync_copy(zbi, dofs_o.at[padidx])
    pltpu.sync_copy(zbf, vals_o.at[padidx])

  def tblz(i, _):
    tblv[pl.ds(i * 16, 16)] = jnp.zeros((16,), jnp.int32)
    return 0
  lax.fori_loop(0, 16, tblz, 0)
  plsc.store_scatter(tblv, [io * 8], s0)
  plsc.store_scatter(tblv, [io * 8 + 1], p0)
  plsc.store_scatter(tblv, [io * 8 + 128], s1)
  plsc.store_scatter(tblv, [io * 8 + 129], p1)
  def tbl_dma(b, _):
    pltpu.sync_copy(tblv.at[pl.ds(b * 8, 8)],
                    tbl_o.at[pl.ds(b * NT * 8 + wid * 8, 8)])
    return 0
  lax.fori_loop(0, NB, tbl_dma, 0)


def _partition(dst, src, val):
  k = pl.kernel(
      _partition_body,
      out_type=(
          jax.ShapeDtypeStruct((EPAD,), jnp.int32),
          jax.ShapeDtypeStruct((EPAD,), jnp.int32),
          jax.ShapeDtypeStruct((EPAD,), jnp.float32),
          jax.ShapeDtypeStruct((NB * NT * 8,), jnp.int32),
      ),
      mesh=_mesh(),
      compiler_params=pltpu.CompilerParams(needs_layout_passes=False),
      scratch_types=(
          pltpu.VMEM((CH1,), jnp.int32),
          pltpu.VMEM((CH1,), jnp.int32),
          pltpu.VMEM((CH1,), jnp.float32),
          pltpu.VMEM((CH1,), jnp.int32),
          pltpu.VMEM((CH1,), jnp.int32),
          pltpu.VMEM((CH1,), jnp.int32),
          pltpu.VMEM((CH1,), jnp.float32),
          pltpu.VMEM((NB,), jnp.int32),
          pltpu.VMEM((NB,), jnp.int32),
          pltpu.VMEM((NB,), jnp.int32),
          pltpu.VMEM((256,), jnp.int32),
          pltpu.VMEM((16,), jnp.int32),
          pltpu.VMEM((512,), jnp.int32),
          pltpu.VMEM((512,), jnp.int32),
          pltpu.VMEM((512,), jnp.float32),
      ),
  )
  return k(dst, src, val)


def _propagate_body(ego, srcs, dofs, vals, tbl, out,
                    tblsm, srcb, dofb, valb, rows, accf, sem):
  wid = _wid()
  io = _io()
  pltpu.sync_copy(tbl.at[pl.ds(wid * NT * 8, NT * 8)],
                  tblsm.at[pl.ds(0, NT * 8)])

  zf = jnp.zeros((16,), jnp.float32)
  def zacc(i, _):
    for k in range(8):
      accf[pl.ds((i * 8 + k) * 16, 16)] = zf
    return 0
  lax.fori_loop(0, ACCW // 128, zacc, 0)

  def seg_body(st, _):
    tv = tblsm[pl.ds(st * 8, 16)]
    off = pl.multiple_of(tv[0], 16)
    cnt = tv[1]
    nch = lax.shift_right_logical(cnt + (KB2 - 1), 9)
    def ch_body(ch, _):
      coff = off + ch * KB2
      c1 = pltpu.async_copy(srcs.at[pl.ds(coff, KB2)], srcb, sem)
      c2 = pltpu.async_copy(dofs.at[pl.ds(coff, KB2)], dofb, sem)
      c3 = pltpu.async_copy(vals.at[pl.ds(coff, KB2)], valb, sem)
      c1.wait()
      c2.wait()
      c3.wait()
      pltpu.async_copy(ego.at[srcb], rows, sem).wait()
      ng = lax.shift_right_logical(
          jnp.minimum(KB2, cnt - ch * KB2), 4)
      def g_body(g, _):
        e16 = g * 16 + io
        v = valb[pl.ds(g * 16, 16)]
        ao = dofb[pl.ds(g * 16, 16)]
        for c in range(EMB):
          cv = plsc.load_gather(rows, [e16, jnp.full((16,), c, jnp.int32)])
          plsc.addupdate_scatter(accf, [ao + c], cv * v)
        return 0
      lax.fori_loop(0, ng, g_body, 0)
      return 0
    lax.fori_loop(0, nch, ch_body, 0)
    return 0
  lax.fori_loop(0, NT, seg_body, 0)

  def relu_body(i, _):
    for k in range(8):
      sl = pl.ds((i * 8 + k) * 16, 16)
      accf[sl] = jnp.maximum(accf[sl], 0.0)
    return 0
  lax.fori_loop(0, ACCW // 128, relu_body, 0)

  @pl.when(wid < NT - 1)
  def _():
    pltpu.sync_copy(accf, out.at[pl.ds(wid * ACCW, ACCW)])
  @pl.when(wid == NT - 1)
  def _():
    tailw = (N_NODES - (NT - 1) * RANGE) * EMB
    pltpu.sync_copy(accf.at[pl.ds(0, tailw)],
                    out.at[pl.ds((NT - 1) * ACCW, tailw)])


def _propagate(ego2d, srcs, dofs, vals, tbl):
  k = pl.kernel(
      _propagate_body,
      out_type=jax.ShapeDtypeStruct((N_NODES * EMB,), jnp.float32),
      mesh=_mesh(),
      compiler_params=pltpu.CompilerParams(
          needs_layout_passes=False, use_tc_tiling_on_sc=False),
      scratch_types=(
          pltpu.VMEM((NT * 8 + 16,), jnp.int32),
          pltpu.VMEM((KB2,), jnp.int32),
          pltpu.VMEM((KB2,), jnp.int32),
          pltpu.VMEM((KB2,), jnp.float32),
          pltpu.VMEM((KB2, EMB), jnp.float32),
          pltpu.VMEM((ACCW,), jnp.float32),
          pltpu.SemaphoreType.DMA,
      ),
  )
  return k(ego2d, srcs, dofs, vals, tbl)


def _mean_kernel(a, b, c, d, o):
  o[...] = 0.25 * (a[...] + b[...] + c[...] + d[...])


def _mean4(a, b, c, d):
  return pl.pallas_call(
      _mean_kernel,
      out_shape=jax.ShapeDtypeStruct((25000, 128), jnp.float32),
      grid=(25,),
      in_specs=[pl.BlockSpec((1000, 128), lambda i: (i, 0))] * 4,
      out_specs=pl.BlockSpec((1000, 128), lambda i: (i, 0)),
  )(a, b, c, d)


def kernel(user_emb, item_emb, adj_values, adj_indices):
  ego0 = jnp.concatenate([user_emb, item_emb], axis=0)
  dst = adj_indices[0]
  src = adj_indices[1]
  srcs, dofs, vals, tbl = _partition(dst, src, adj_values)
  egos = [ego0.reshape(-1)]
  e2d = ego0
  for _ in range(3):
    ef = _propagate(e2d, srcs, dofs, vals, tbl)
    egos.append(ef)
    e2d = ef.reshape(N_NODES, EMB)
  m = _mean4(*[x.reshape(25000, 128) for x in egos])
  m = m.reshape(N_NODES, EMB)
  return (m[:50000], m[50000:])


# speedup vs baseline: 2.8745x; 2.0340x over previous
import functools

import jax
import jax.numpy as jnp
from jax import lax
from jax.experimental import pallas as pl
from jax.experimental.pallas import tpu as pltpu
from jax.experimental.pallas import tpu_sc as plsc

N_NODES = 100000
EMB = 32
E = 1600000
NT = 32
PT = E // NT
NB = 32
RANGE = 3200
RSZ = PT + NB * 16
EPAD = NT * RSZ + 528
DUMP = EPAD - 16
CH1 = 2000
KB2 = 512
ACCW = RANGE * EMB

_mesh = functools.partial(
    plsc.VectorSubcoreMesh, core_axis_name="c", subcore_axis_name="s")


def _wid():
  return lax.axis_index("s") * 2 + lax.axis_index("c")


def _bucket(d):
  q = lax.shift_right_logical(d, 7)
  return lax.shift_right_logical(q * 5243, 17)


def _io():
  return lax.iota(jnp.int32, 16)


def _ranks(sb, sbuf):
  io = _io()
  sbuf[...] = sb
  prev = plsc.load_gather(sbuf, [jnp.maximum(io - 1, 0)])
  nxt = plsc.load_gather(sbuf, [jnp.minimum(io + 1, 15)])
  first = jnp.logical_or(io == 0, sb != prev)
  is_end = jnp.logical_or(io == 15, sb != nxt)
  start = plsc.cummax(jnp.where(first, io, 0))
  rank = io - start
  return rank, is_end


STG = 160


def _partition_body(dst, src, val, srcs_o, dofs_o, vals_o, tbl_o,
                    dstb, srcb, valb, bkb, dofb,
                    hist, tblv, sbuf, stg_s, stg_d, stg_v,
                    zb16i, zb16f, cntS, gposS, sem):
  wid = _wid()
  slab = wid * PT
  regbase = wid * RSZ
  io = _io()

  hist[pl.ds(0, 16)] = jnp.zeros((16,), jnp.int32)
  hist[pl.ds(16, 16)] = jnp.zeros((16,), jnp.int32)
  zb16i[pl.ds(0, 16)] = jnp.zeros((16,), jnp.int32)
  zb16f[pl.ds(0, 16)] = jnp.zeros((16,), jnp.float32)

  def p1_chunk(c, _):
    pltpu.sync_copy(dst.at[pl.ds(slab + c * CH1, CH1)], dstb)
    def p1_vreg(i, _):
      d = dstb[pl.ds(i * 16, 16)]
      b = _bucket(d)
      sb, _ = plsc.sort_key_val(b, io)
      rank, is_end = _ranks(sb, sbuf)
      h = plsc.load_gather(hist, [sb])
      plsc.store_scatter(hist, [sb], h + rank + 1, mask=is_end)
      return 0
    lax.fori_loop(0, CH1 // 16, p1_vreg, 0)
    return 0
  lax.fori_loop(0, PT // CH1, p1_chunk, 0)

  h0 = hist[pl.ds(0, 16)]
  h1 = hist[pl.ds(16, 16)]
  p0 = jnp.bitwise_and(h0 + 15, -16)
  p1 = jnp.bitwise_and(h1 + 15, -16)
  c0 = plsc.cumsum(p0)
  c1 = plsc.cumsum(p1)
  tot0 = jnp.max(c0)
  s0 = regbase + (c0 - p0)
  s1 = regbase + (c1 - p1) + tot0

  for b in range(16):
    gposS[b] = s0[b]
    gposS[16 + b] = s1[b]
  for b in range(32):
    cntS[b] = 0

  def p2_chunk(c, _):
    coff = slab + c * CH1
    pltpu.sync_copy(dst.at[pl.ds(coff, CH1)], dstb)
    pltpu.sync_copy(src.at[pl.ds(coff, CH1)], srcb)
    pltpu.sync_copy(val.at[pl.ds(coff, CH1)], valb)
    def prep(i, _):
      d = dstb[pl.ds(i * 16, 16)]
      b = _bucket(d)
      bkb[pl.ds(i * 16, 16)] = b
      dofb[pl.ds(i * 16, 16)] = lax.shift_left(d - b * RANGE, 5)
      return 0
    lax.fori_loop(0, CH1 // 16, prep, 0)
    for b in range(NB):
      sbase = b * STG
      def v_body(i, cnt):
        sl = pl.ds(i * 16, 16)
        m = bkb[sl] == b
        n = plsc.all_reduce_population_count(m)[0]
        off = sbase + cnt
        plsc.store_compressed(stg_s.at[pl.ds(off, 16)], srcb[sl], mask=m)
        plsc.store_compressed(stg_d.at[pl.ds(off, 16)], dofb[sl], mask=m)
        plsc.store_compressed(stg_v.at[pl.ds(off, 16)], valb[sl], mask=m)
        cnt2 = cnt + n
        @pl.when(cnt2 >= 128)
        def _():
          g = pl.multiple_of(gposS[b], 16)
          d1 = pltpu.async_copy(stg_s.at[pl.ds(sbase, 128)],
                                srcs_o.at[pl.ds(g, 128)], sem)
          d2 = pltpu.async_copy(stg_d.at[pl.ds(sbase, 128)],
                                dofs_o.at[pl.ds(g, 128)], sem)
          d3 = pltpu.async_copy(stg_v.at[pl.ds(sbase, 128)],
                                vals_o.at[pl.ds(g, 128)], sem)
          d1.wait()
          d2.wait()
          d3.wait()
          stg_s[pl.ds(sbase, 16)] = stg_s[pl.ds(sbase + 128, 16)]
          stg_d[pl.ds(sbase, 16)] = stg_d[pl.ds(sbase + 128, 16)]
          stg_v[pl.ds(sbase, 16)] = stg_v[pl.ds(sbase + 128, 16)]
          gposS[b] = g + 128
        return jnp.where(cnt2 >= 128, cnt2 - 128, cnt2)
      cntS[b] = lax.fori_loop(0, CH1 // 16, v_body, cntS[b])
    return 0
  lax.fori_loop(0, PT // CH1, p2_chunk, 0)

  for b in range(NB):
    sbase = b * STG
    cnt = cntS[b]
    stg_s[pl.ds(sbase + cnt, 16)] = jnp.zeros((16,), jnp.int32)
    stg_d[pl.ds(sbase + cnt, 16)] = jnp.zeros((16,), jnp.int32)
    stg_v[pl.ds(sbase + cnt, 16)] = jnp.zeros((16,), jnp.float32)
    g0 = pl.multiple_of(gposS[b], 16)
    nfl = lax.shift_right_logical(cnt + 15, 4)
    def dr(i, _):
      o1 = pl.ds(sbase + i * 16, 16)
      o2 = pl.ds(g0 + i * 16, 16)
      d1 = pltpu.async_copy(stg_s.at[o1], srcs_o.at[o2], sem)
      d2 = pltpu.async_copy(stg_d.at[o1], dofs_o.at[o2], sem)
      d3 = pltpu.async_copy(stg_v.at[o1], vals_o.at[o2], sem)
      d1.wait()
      d2.wait()
      d3.wait()
      return 0
    lax.fori_loop(0, nfl, dr, 0)

  regend = regbase + jnp.max(c1) + tot0
  cap = jnp.where(wid == NT - 1, regbase + RSZ + 512, regbase + RSZ)
  nz = lax.shift_right_logical(cap - regend, 4)
  def z_body(i, _):
    o = pl.ds(pl.multiple_of(regend + i * 16, 16), 16)
    d1 = pltpu.async_copy(zb16i, srcs_o.at[o], sem)
    d2 = pltpu.async_copy(zb16i, dofs_o.at[o], sem)
    d3 = pltpu.async_copy(zb16f, vals_o.at[o], sem)
    d1.wait()
    d2.wait()
    d3.wait()
    return 0
  lax.fori_loop(0, nz, z_body, 0)

  def tblz(i, _):
    tblv[pl.ds(i * 16, 16)] = jnp.zeros((16,), jnp.int32)
    return 0
  lax.fori_loop(0, 16, tblz, 0)
  plsc.store_scatter(tblv, [io * 8], s0)
  plsc.store_scatter(tblv, [io * 8 + 1], p0)
  plsc.store_scatter(tblv, [io * 8 + 128], s1)
  plsc.store_scatter(tblv, [io * 8 + 129], p1)
  def tbl_dma(b, _):
    pltpu.sync_copy(tblv.at[pl.ds(b * 8, 8)],
                    tbl_o.at[pl.ds(b * NT * 8 + wid * 8, 8)])
    return 0
  lax.fori_loop(0, NB, tbl_dma, 0)


def _partition(dst, src, val):
  k = pl.kernel(
      _partition_body,
      out_type=(
          jax.ShapeDtypeStruct((EPAD,), jnp.int32),
          jax.ShapeDtypeStruct((EPAD,), jnp.int32),
          jax.ShapeDtypeStruct((EPAD,), jnp.float32),
          jax.ShapeDtypeStruct((NB * NT * 8,), jnp.int32),
      ),
      mesh=_mesh(),
      compiler_params=pltpu.CompilerParams(needs_layout_passes=False),
      scratch_types=(
          pltpu.VMEM((CH1,), jnp.int32),
          pltpu.VMEM((CH1,), jnp.int32),
          pltpu.VMEM((CH1,), jnp.float32),
          pltpu.VMEM((CH1,), jnp.int32),
          pltpu.VMEM((CH1,), jnp.int32),
          pltpu.VMEM((NB,), jnp.int32),
          pltpu.VMEM((256,), jnp.int32),
          pltpu.VMEM((16,), jnp.int32),
          pltpu.VMEM((NB * STG,), jnp.int32),
          pltpu.VMEM((NB * STG,), jnp.int32),
          pltpu.VMEM((NB * STG,), jnp.float32),
          pltpu.VMEM((16,), jnp.int32),
          pltpu.VMEM((16,), jnp.float32),
          pltpu.SMEM((NB,), jnp.int32),
          pltpu.SMEM((NB,), jnp.int32),
          pltpu.SemaphoreType.DMA,
      ),
  )
  return k(dst, src, val)


def _propagate_body(ego, srcs, dofs, vals, tbl, out,
                    tblsm, srcb, dofb, valb, rows, accf, sem):
  wid = _wid()
  io = _io()
  pltpu.sync_copy(tbl.at[pl.ds(wid * NT * 8, NT * 8)],
                  tblsm.at[pl.ds(0, NT * 8)])

  zf = jnp.zeros((16,), jnp.float32)
  def zacc(i, _):
    for k in range(8):
      accf[pl.ds((i * 8 + k) * 16, 16)] = zf
    return 0
  lax.fori_loop(0, ACCW // 128, zacc, 0)

  def seg_body(st, _):
    tv = tblsm[pl.ds(st * 8, 16)]
    off = pl.multiple_of(tv[0], 16)
    cnt = tv[1]
    nch = lax.shift_right_logical(cnt + (KB2 - 1), 9)
    def ch_body(ch, _):
      coff = off + ch * KB2
      c1 = pltpu.async_copy(srcs.at[pl.ds(coff, KB2)], srcb, sem)
      c2 = pltpu.async_copy(dofs.at[pl.ds(coff, KB2)], dofb, sem)
      c3 = pltpu.async_copy(vals.at[pl.ds(coff, KB2)], valb, sem)
      c1.wait()
      c2.wait()
      c3.wait()
      pltpu.async_copy(ego.at[srcb], rows, sem).wait()
      ng = lax.shift_right_logical(
          jnp.minimum(KB2, cnt - ch * KB2), 4)
      @plsc.parallel_loop(0, ng)
      def _(g):
        e16 = g * 16 + io
        v = valb[pl.ds(g * 16, 16)]
        ao = dofb[pl.ds(g * 16, 16)]
        for c in range(EMB):
          cv = plsc.load_gather(rows, [e16, jnp.full((16,), c, jnp.int32)])
          plsc.addupdate_scatter(accf, [ao + c], cv * v)
      return 0
    lax.fori_loop(0, nch, ch_body, 0)
    return 0
  lax.fori_loop(0, NT, seg_body, 0)

  def relu_body(i, _):
    for k in range(8):
      sl = pl.ds((i * 8 + k) * 16, 16)
      accf[sl] = jnp.maximum(accf[sl], 0.0)
    return 0
  lax.fori_loop(0, ACCW // 128, relu_body, 0)

  @pl.when(wid < NT - 1)
  def _():
    pltpu.sync_copy(accf, out.at[pl.ds(wid * ACCW, ACCW)])
  @pl.when(wid == NT - 1)
  def _():
    tailw = (N_NODES - (NT - 1) * RANGE) * EMB
    pltpu.sync_copy(accf.at[pl.ds(0, tailw)],
                    out.at[pl.ds((NT - 1) * ACCW, tailw)])


def _propagate(ego2d, srcs, dofs, vals, tbl):
  k = pl.kernel(
      _propagate_body,
      out_type=jax.ShapeDtypeStruct((N_NODES * EMB,), jnp.float32),
      mesh=_mesh(),
      compiler_params=pltpu.CompilerParams(
          needs_layout_passes=False, use_tc_tiling_on_sc=False),
      scratch_types=(
          pltpu.VMEM((NT * 8 + 16,), jnp.int32),
          pltpu.VMEM((KB2,), jnp.int32),
          pltpu.VMEM((KB2,), jnp.int32),
          pltpu.VMEM((KB2,), jnp.float32),
          pltpu.VMEM((KB2, EMB), jnp.float32),
          pltpu.VMEM((ACCW,), jnp.float32),
          pltpu.SemaphoreType.DMA,
      ),
  )
  return k(ego2d, srcs, dofs, vals, tbl)


def _mean_kernel(a, b, c, d, o):
  o[...] = 0.25 * (a[...] + b[...] + c[...] + d[...])


def _mean4(a, b, c, d):
  return pl.pallas_call(
      _mean_kernel,
      out_shape=jax.ShapeDtypeStruct((25000, 128), jnp.float32),
      grid=(25,),
      in_specs=[pl.BlockSpec((1000, 128), lambda i: (i, 0))] * 4,
      out_specs=pl.BlockSpec((1000, 128), lambda i: (i, 0)),
  )(a, b, c, d)


def kernel(user_emb, item_emb, adj_values, adj_indices):
  ego0 = jnp.concatenate([user_emb, item_emb], axis=0)
  dst = adj_indices[0]
  src = adj_indices[1]
  srcs, dofs, vals, tbl = _partition(dst, src, adj_values)
  egos = [ego0.reshape(-1)]
  e2d = ego0
  for _ in range(3):
    ef = _propagate(e2d, srcs, dofs, vals, tbl)
    egos.append(ef)
    e2d = ef.reshape(N_NODES, EMB)
  m = _mean4(*[x.reshape(25000, 128) for x in egos])
  m = m.reshape(N_NODES, EMB)
  return (m[:50000], m[50000:])


# speedup vs baseline: 4.0520x; 1.4096x over previous
import functools

import jax
import jax.numpy as jnp
from jax import lax
from jax.experimental import pallas as pl
from jax.experimental.pallas import tpu as pltpu
from jax.experimental.pallas import tpu_sc as plsc

N_NODES = 100000
EMB = 32
E = 1600000
NT = 32
PT = E // NT
NB = 32
RANGE = 3200
RSZ = PT + NB * 16
EPAD = NT * RSZ + 528
DUMP = EPAD - 16
CH1 = 2000
KB2 = 512
ACCW = RANGE * EMB

_mesh = functools.partial(
    plsc.VectorSubcoreMesh, core_axis_name="c", subcore_axis_name="s")


def _wid():
  return lax.axis_index("s") * 2 + lax.axis_index("c")


def _bucket(d):
  q = lax.shift_right_logical(d, 7)
  return lax.shift_right_logical(q * 5243, 17)


def _io():
  return lax.iota(jnp.int32, 16)


def _ranks(sb, sbuf):
  io = _io()
  sbuf[...] = sb
  prev = plsc.load_gather(sbuf, [jnp.maximum(io - 1, 0)])
  nxt = plsc.load_gather(sbuf, [jnp.minimum(io + 1, 15)])
  first = jnp.logical_or(io == 0, sb != prev)
  is_end = jnp.logical_or(io == 15, sb != nxt)
  start = plsc.cummax(jnp.where(first, io, 0))
  rank = io - start
  return rank, is_end


STG = 160


def _partition_body(dst, src, val, srcs_o, dofs_o, vals_o, tbl_o,
                    dstb, srcb, valb, cntv,
                    hist, tblv, sbuf, stg_s, stg_d, stg_v,
                    zb16i, zb16f, gposS, sem):
  wid = _wid()
  slab = wid * PT
  regbase = wid * RSZ
  io = _io()

  hist[pl.ds(0, 16)] = jnp.zeros((16,), jnp.int32)
  hist[pl.ds(16, 16)] = jnp.zeros((16,), jnp.int32)
  zb16i[pl.ds(0, 16)] = jnp.zeros((16,), jnp.int32)
  zb16f[pl.ds(0, 16)] = jnp.zeros((16,), jnp.float32)

  def p1_chunk(c, _):
    pltpu.sync_copy(dst.at[pl.ds(slab + c * CH1, CH1)], dstb)
    def p1_vreg(i, _):
      d = dstb[pl.ds(i * 16, 16)]
      b = _bucket(d)
      sb, _ = plsc.sort_key_val(b, io)
      rank, is_end = _ranks(sb, sbuf)
      h = plsc.load_gather(hist, [sb])
      plsc.store_scatter(hist, [sb], h + rank + 1, mask=is_end)
      return 0
    lax.fori_loop(0, CH1 // 16, p1_vreg, 0)
    return 0
  lax.fori_loop(0, PT // CH1, p1_chunk, 0)

  h0 = hist[pl.ds(0, 16)]
  h1 = hist[pl.ds(16, 16)]
  p0 = jnp.bitwise_and(h0 + 15, -16)
  p1 = jnp.bitwise_and(h1 + 15, -16)
  c0 = plsc.cumsum(p0)
  c1 = plsc.cumsum(p1)
  tot0 = jnp.max(c0)
  s0 = regbase + (c0 - p0)
  s1 = regbase + (c1 - p1) + tot0

  for b in range(16):
    gposS[b] = s0[b]
    gposS[16 + b] = s1[b]

  cntv[pl.ds(0, 16)] = jnp.zeros((16,), jnp.int32)
  cntv[pl.ds(16, 16)] = jnp.zeros((16,), jnp.int32)

  def p2_chunk(c, _):
    coff = slab + c * CH1
    pltpu.sync_copy(dst.at[pl.ds(coff, CH1)], dstb)
    pltpu.sync_copy(src.at[pl.ds(coff, CH1)], srcb)
    pltpu.sync_copy(val.at[pl.ds(coff, CH1)], valb)
    def v_body(i, _):
      d = dstb[pl.ds(i * 16, 16)]
      b = _bucket(d)
      sb, lanes = plsc.sort_key_val(b, i * 16 + io)
      rank, is_end = _ranks(sb, sbuf)
      base = plsc.load_gather(cntv, [sb])
      pos = base + rank
      newc = pos + 1
      plsc.store_scatter(cntv, [sb], newc, mask=is_end)
      d_s = plsc.load_gather(dstb, [lanes])
      s_s = plsc.load_gather(srcb, [lanes])
      v_s = plsc.load_gather(valb, [lanes])
      doff_s = lax.shift_left(d_s - sb * RANGE, 5)
      addr = sb * STG + pos
      plsc.store_scatter(stg_s, [addr], s_s)
      plsc.store_scatter(stg_d, [addr], doff_s)
      plsc.store_scatter(stg_v, [addr], v_s)
      @pl.when(jnp.max(newc) >= 128)
      def _():
        cl = cntv[pl.ds(0, 16)]
        ch = cntv[pl.ds(16, 16)]
        for b2 in range(NB):
          cb = cl[b2] if b2 < 16 else ch[b2 - 16]
          sbase2 = b2 * STG
          @pl.when(cb >= 128)
          def _():
            g = pl.multiple_of(gposS[b2], 16)
            d1 = pltpu.async_copy(stg_s.at[pl.ds(sbase2, 128)],
                                  srcs_o.at[pl.ds(g, 128)], sem)
            d2 = pltpu.async_copy(stg_d.at[pl.ds(sbase2, 128)],
                                  dofs_o.at[pl.ds(g, 128)], sem)
            d3 = pltpu.async_copy(stg_v.at[pl.ds(sbase2, 128)],
                                  vals_o.at[pl.ds(g, 128)], sem)
            d1.wait()
            d2.wait()
            d3.wait()
            stg_s[pl.ds(sbase2, 16)] = stg_s[pl.ds(sbase2 + 128, 16)]
            stg_d[pl.ds(sbase2, 16)] = stg_d[pl.ds(sbase2 + 128, 16)]
            stg_v[pl.ds(sbase2, 16)] = stg_v[pl.ds(sbase2 + 128, 16)]
            gposS[b2] = g + 128
            plsc.store_scatter(cntv, [jnp.full((16,), b2, jnp.int32)],
                               jnp.full((16,), cb - 128, jnp.int32),
                               mask=io == 0)
      return 0
    lax.fori_loop(0, CH1 // 16, v_body, 0)
    return 0
  lax.fori_loop(0, PT // CH1, p2_chunk, 0)

  cl = cntv[pl.ds(0, 16)]
  ch = cntv[pl.ds(16, 16)]
  for b in range(NB):
    sbase = b * STG
    cnt = cl[b] if b < 16 else ch[b - 16]
    stg_s[pl.ds(sbase + cnt, 16)] = jnp.zeros((16,), jnp.int32)
    stg_d[pl.ds(sbase + cnt, 16)] = jnp.zeros((16,), jnp.int32)
    stg_v[pl.ds(sbase + cnt, 16)] = jnp.zeros((16,), jnp.float32)
    g0 = pl.multiple_of(gposS[b], 16)
    nfl = lax.shift_right_logical(cnt + 15, 4)
    def dr(i, _):
      o1 = pl.ds(sbase + i * 16, 16)
      o2 = pl.ds(g0 + i * 16, 16)
      d1 = pltpu.async_copy(stg_s.at[o1], srcs_o.at[o2], sem)
      d2 = pltpu.async_copy(stg_d.at[o1], dofs_o.at[o2], sem)
      d3 = pltpu.async_copy(stg_v.at[o1], vals_o.at[o2], sem)
      d1.wait()
      d2.wait()
      d3.wait()
      return 0
    lax.fori_loop(0, nfl, dr, 0)

  regend = regbase + jnp.max(c1) + tot0
  cap = jnp.where(wid == NT - 1, regbase + RSZ + 512, regbase + RSZ)
  nz = lax.shift_right_logical(cap - regend, 4)
  def z_body(i, _):
    o = pl.ds(pl.multiple_of(regend + i * 16, 16), 16)
    d1 = pltpu.async_copy(zb16i, srcs_o.at[o], sem)
    d2 = pltpu.async_copy(zb16i, dofs_o.at[o], sem)
    d3 = pltpu.async_copy(zb16f, vals_o.at[o], sem)
    d1.wait()
    d2.wait()
    d3.wait()
    return 0
  lax.fori_loop(0, nz, z_body, 0)

  def tblz(i, _):
    tblv[pl.ds(i * 16, 16)] = jnp.zeros((16,), jnp.int32)
    return 0
  lax.fori_loop(0, 16, tblz, 0)
  plsc.store_scatter(tblv, [io * 8], s0)
  plsc.store_scatter(tblv, [io * 8 + 1], p0)
  plsc.store_scatter(tblv, [io * 8 + 128], s1)
  plsc.store_scatter(tblv, [io * 8 + 129], p1)
  def tbl_dma(b, _):
    pltpu.sync_copy(tblv.at[pl.ds(b * 8, 8)],
                    tbl_o.at[pl.ds(b * NT * 8 + wid * 8, 8)])
    return 0
  lax.fori_loop(0, NB, tbl_dma, 0)


def _partition(dst, src, val):
  k = pl.kernel(
      _partition_body,
      out_type=(
          jax.ShapeDtypeStruct((EPAD,), jnp.int32),
          jax.ShapeDtypeStruct((EPAD,), jnp.int32),
          jax.ShapeDtypeStruct((EPAD,), jnp.float32),
          jax.ShapeDtypeStruct((NB * NT * 8,), jnp.int32),
      ),
      mesh=_mesh(),
      compiler_params=pltpu.CompilerParams(needs_layout_passes=False),
      scratch_types=(
          pltpu.VMEM((CH1,), jnp.int32),
          pltpu.VMEM((CH1,), jnp.int32),
          pltpu.VMEM((CH1,), jnp.float32),
          pltpu.VMEM((NB,), jnp.int32),
          pltpu.VMEM((NB,), jnp.int32),
          pltpu.VMEM((256,), jnp.int32),
          pltpu.VMEM((16,), jnp.int32),
          pltpu.VMEM((NB * STG,), jnp.int32),
          pltpu.VMEM((NB * STG,), jnp.int32),
          pltpu.VMEM((NB * STG,), jnp.float32),
          pltpu.VMEM((16,), jnp.int32),
          pltpu.VMEM((16,), jnp.float32),
          pltpu.SMEM((NB,), jnp.int32),
          pltpu.SemaphoreType.DMA,
      ),
  )
  return k(dst, src, val)


def _propagate_body(ego, srcs, dofs, vals, tbl, out,
                    tblsm, srcb, dofb, valb, rows, accf, sem):
  wid = _wid()
  io = _io()
  pltpu.sync_copy(tbl.at[pl.ds(wid * NT * 8, NT * 8)],
                  tblsm.at[pl.ds(0, NT * 8)])

  zf = jnp.zeros((16,), jnp.float32)
  def zacc(i, _):
    for k in range(8):
      accf[pl.ds((i * 8 + k) * 16, 16)] = zf
    return 0
  lax.fori_loop(0, ACCW // 128, zacc, 0)

  def seg_body(st, _):
    tv = tblsm[pl.ds(st * 8, 16)]
    off = pl.multiple_of(tv[0], 16)
    cnt = tv[1]
    nch = lax.shift_right_logical(cnt + (KB2 - 1), 9)
    def ch_body(ch, _):
      coff = off + ch * KB2
      c1 = pltpu.async_copy(srcs.at[pl.ds(coff, KB2)], srcb, sem)
      c2 = pltpu.async_copy(dofs.at[pl.ds(coff, KB2)], dofb, sem)
      c3 = pltpu.async_copy(vals.at[pl.ds(coff, KB2)], valb, sem)
      c1.wait()
      c2.wait()
      c3.wait()
      pltpu.async_copy(ego.at[srcb], rows, sem).wait()
      ng = lax.shift_right_logical(
          jnp.minimum(KB2, cnt - ch * KB2), 4)
      @plsc.parallel_loop(0, ng)
      def _(g):
        e16 = g * 16 + io
        v = valb[pl.ds(g * 16, 16)]
        ao = dofb[pl.ds(g * 16, 16)]
        for c in range(EMB):
          cv = plsc.load_gather(rows, [e16, jnp.full((16,), c, jnp.int32)])
          plsc.addupdate_scatter(accf, [ao + c], cv * v)
      return 0
    lax.fori_loop(0, nch, ch_body, 0)
    return 0
  lax.fori_loop(0, NT, seg_body, 0)

  def relu_body(i, _):
    for k in range(8):
      sl = pl.ds((i * 8 + k) * 16, 16)
      accf[sl] = jnp.maximum(accf[sl], 0.0)
    return 0
  lax.fori_loop(0, ACCW // 128, relu_body, 0)

  @pl.when(wid < NT - 1)
  def _():
    pltpu.sync_copy(accf, out.at[pl.ds(wid * ACCW, ACCW)])
  @pl.when(wid == NT - 1)
  def _():
    tailw = (N_NODES - (NT - 1) * RANGE) * EMB
    pltpu.sync_copy(accf.at[pl.ds(0, tailw)],
                    out.at[pl.ds((NT - 1) * ACCW, tailw)])


def _propagate(ego2d, srcs, dofs, vals, tbl):
  k = pl.kernel(
      _propagate_body,
      out_type=jax.ShapeDtypeStruct((N_NODES * EMB,), jnp.float32),
      mesh=_mesh(),
      compiler_params=pltpu.CompilerParams(
          needs_layout_passes=False, use_tc_tiling_on_sc=False),
      scratch_types=(
          pltpu.VMEM((NT * 8 + 16,), jnp.int32),
          pltpu.VMEM((KB2,), jnp.int32),
          pltpu.VMEM((KB2,), jnp.int32),
          pltpu.VMEM((KB2,), jnp.float32),
          pltpu.VMEM((KB2, EMB), jnp.float32),
          pltpu.VMEM((ACCW,), jnp.float32),
          pltpu.SemaphoreType.DMA,
      ),
  )
  return k(ego2d, srcs, dofs, vals, tbl)


def _mean_kernel(a, b, c, d, o):
  o[...] = 0.25 * (a[...] + b[...] + c[...] + d[...])


def _mean4(a, b, c, d):
  return pl.pallas_call(
      _mean_kernel,
      out_shape=jax.ShapeDtypeStruct((25000, 128), jnp.float32),
      grid=(25,),
      in_specs=[pl.BlockSpec((1000, 128), lambda i: (i, 0))] * 4,
      out_specs=pl.BlockSpec((1000, 128), lambda i: (i, 0)),
  )(a, b, c, d)


def kernel(user_emb, item_emb, adj_values, adj_indices):
  ego0 = jnp.concatenate([user_emb, item_emb], axis=0)
  dst = adj_indices[0]
  src = adj_indices[1]
  srcs, dofs, vals, tbl = _partition(dst, src, adj_values)
  egos = [ego0.reshape(-1)]
  e2d = ego0
  for _ in range(3):
    ef = _propagate(e2d, srcs, dofs, vals, tbl)
    egos.append(ef)
    e2d = ef.reshape(N_NODES, EMB)
  m = _mean4(*[x.reshape(25000, 128) for x in egos])
  m = m.reshape(N_NODES, EMB)
  return (m[:50000], m[50000:])


# speedup vs baseline: 4.2248x; 1.0427x over previous
import functools

import jax
import jax.numpy as jnp
from jax import lax
from jax.experimental import pallas as pl
from jax.experimental.pallas import tpu as pltpu
from jax.experimental.pallas import tpu_sc as plsc

N_NODES = 100000
EMB = 32
E = 1600000
NT = 32
PT = E // NT
NB = 32
RANGE = 3200
RSZ = PT + NB * 16
EPAD = NT * RSZ + 528
DUMP = EPAD - 16
CH1 = 2000
KBC = 256
ACCW = RANGE * EMB

_mesh = functools.partial(
    plsc.VectorSubcoreMesh, core_axis_name="c", subcore_axis_name="s")


def _wid():
  return lax.axis_index("s") * 2 + lax.axis_index("c")


def _bucket(d):
  q = lax.shift_right_logical(d, 7)
  return lax.shift_right_logical(q * 5243, 17)


def _io():
  return lax.iota(jnp.int32, 16)


def _ranks(sb, sbuf):
  io = _io()
  sbuf[...] = sb
  prev = plsc.load_gather(sbuf, [jnp.maximum(io - 1, 0)])
  nxt = plsc.load_gather(sbuf, [jnp.minimum(io + 1, 15)])
  first = jnp.logical_or(io == 0, sb != prev)
  is_end = jnp.logical_or(io == 15, sb != nxt)
  start = plsc.cummax(jnp.where(first, io, 0))
  rank = io - start
  return rank, is_end


STG = 160


def _partition_body(dst, src, val, srcs_o, dofs_o, vals_o, tbl_o,
                    dstb, srcb, valb, cntv,
                    hist, tblv, sbuf, stg_s, stg_d, stg_v,
                    zb16i, zb16f, gposS, sem):
  wid = _wid()
  slab = wid * PT
  regbase = wid * RSZ
  io = _io()

  hist[pl.ds(0, 16)] = jnp.zeros((16,), jnp.int32)
  hist[pl.ds(16, 16)] = jnp.zeros((16,), jnp.int32)
  zb16i[pl.ds(0, 16)] = jnp.zeros((16,), jnp.int32)
  zb16f[pl.ds(0, 16)] = jnp.zeros((16,), jnp.float32)

  def p1_chunk(c, _):
    pltpu.sync_copy(dst.at[pl.ds(slab + c * CH1, CH1)], dstb)
    def p1_vreg(i, _):
      d = dstb[pl.ds(i * 16, 16)]
      b = _bucket(d)
      sb, _ = plsc.sort_key_val(b, io)
      rank, is_end = _ranks(sb, sbuf)
      h = plsc.load_gather(hist, [sb])
      plsc.store_scatter(hist, [sb], h + rank + 1, mask=is_end)
      return 0
    lax.fori_loop(0, CH1 // 16, p1_vreg, 0)
    return 0
  lax.fori_loop(0, PT // CH1, p1_chunk, 0)

  h0 = hist[pl.ds(0, 16)]
  h1 = hist[pl.ds(16, 16)]
  p0 = jnp.bitwise_and(h0 + 15, -16)
  p1 = jnp.bitwise_and(h1 + 15, -16)
  c0 = plsc.cumsum(p0)
  c1 = plsc.cumsum(p1)
  tot0 = jnp.max(c0)
  s0 = regbase + (c0 - p0)
  s1 = regbase + (c1 - p1) + tot0

  for b in range(16):
    gposS[b] = s0[b]
    gposS[16 + b] = s1[b]

  cntv[pl.ds(0, 16)] = jnp.zeros((16,), jnp.int32)
  cntv[pl.ds(16, 16)] = jnp.zeros((16,), jnp.int32)

  def p2_chunk(c, _):
    coff = slab + c * CH1
    pltpu.sync_copy(dst.at[pl.ds(coff, CH1)], dstb)
    pltpu.sync_copy(src.at[pl.ds(coff, CH1)], srcb)
    pltpu.sync_copy(val.at[pl.ds(coff, CH1)], valb)
    def v_body(i, _):
      d = dstb[pl.ds(i * 16, 16)]
      b = _bucket(d)
      sb, lanes = plsc.sort_key_val(b, i * 16 + io)
      rank, is_end = _ranks(sb, sbuf)
      base = plsc.load_gather(cntv, [sb])
      pos = base + rank
      newc = pos + 1
      plsc.store_scatter(cntv, [sb], newc, mask=is_end)
      d_s = plsc.load_gather(dstb, [lanes])
      s_s = plsc.load_gather(srcb, [lanes])
      v_s = plsc.load_gather(valb, [lanes])
      doff_s = lax.shift_left(d_s - sb * RANGE, 5)
      addr = sb * STG + pos
      plsc.store_scatter(stg_s, [addr], s_s)
      plsc.store_scatter(stg_d, [addr], doff_s)
      plsc.store_scatter(stg_v, [addr], v_s)
      @pl.when(jnp.max(newc) >= 128)
      def _():
        cl = cntv[pl.ds(0, 16)]
        ch = cntv[pl.ds(16, 16)]
        for b2 in range(NB):
          cb = cl[b2] if b2 < 16 else ch[b2 - 16]
          sbase2 = b2 * STG
          @pl.when(cb >= 128)
          def _():
            g = pl.multiple_of(gposS[b2], 16)
            d1 = pltpu.async_copy(stg_s.at[pl.ds(sbase2, 128)],
                                  srcs_o.at[pl.ds(g, 128)], sem)
            d2 = pltpu.async_copy(stg_d.at[pl.ds(sbase2, 128)],
                                  dofs_o.at[pl.ds(g, 128)], sem)
            d3 = pltpu.async_copy(stg_v.at[pl.ds(sbase2, 128)],
                                  vals_o.at[pl.ds(g, 128)], sem)
            d1.wait()
            d2.wait()
            d3.wait()
            stg_s[pl.ds(sbase2, 16)] = stg_s[pl.ds(sbase2 + 128, 16)]
            stg_d[pl.ds(sbase2, 16)] = stg_d[pl.ds(sbase2 + 128, 16)]
            stg_v[pl.ds(sbase2, 16)] = stg_v[pl.ds(sbase2 + 128, 16)]
            gposS[b2] = g + 128
            plsc.store_scatter(cntv, [jnp.full((16,), b2, jnp.int32)],
                               jnp.full((16,), cb - 128, jnp.int32),
                               mask=io == 0)
      return 0
    lax.fori_loop(0, CH1 // 16, v_body, 0)
    return 0
  lax.fori_loop(0, PT // CH1, p2_chunk, 0)

  cl = cntv[pl.ds(0, 16)]
  ch = cntv[pl.ds(16, 16)]
  for b in range(NB):
    sbase = b * STG
    cnt = cl[b] if b < 16 else ch[b - 16]
    stg_s[pl.ds(sbase + cnt, 16)] = jnp.zeros((16,), jnp.int32)
    stg_d[pl.ds(sbase + cnt, 16)] = jnp.zeros((16,), jnp.int32)
    stg_v[pl.ds(sbase + cnt, 16)] = jnp.zeros((16,), jnp.float32)
    g0 = pl.multiple_of(gposS[b], 16)
    nfl = lax.shift_right_logical(cnt + 15, 4)
    def dr(i, _):
      o1 = pl.ds(sbase + i * 16, 16)
      o2 = pl.ds(g0 + i * 16, 16)
      d1 = pltpu.async_copy(stg_s.at[o1], srcs_o.at[o2], sem)
      d2 = pltpu.async_copy(stg_d.at[o1], dofs_o.at[o2], sem)
      d3 = pltpu.async_copy(stg_v.at[o1], vals_o.at[o2], sem)
      d1.wait()
      d2.wait()
      d3.wait()
      return 0
    lax.fori_loop(0, nfl, dr, 0)

  regend = regbase + jnp.max(c1) + tot0
  cap = jnp.where(wid == NT - 1, regbase + RSZ + 512, regbase + RSZ)
  nz = lax.shift_right_logical(cap - regend, 4)
  def z_body(i, _):
    o = pl.ds(pl.multiple_of(regend + i * 16, 16), 16)
    d1 = pltpu.async_copy(zb16i, srcs_o.at[o], sem)
    d2 = pltpu.async_copy(zb16i, dofs_o.at[o], sem)
    d3 = pltpu.async_copy(zb16f, vals_o.at[o], sem)
    d1.wait()
    d2.wait()
    d3.wait()
    return 0
  lax.fori_loop(0, nz, z_body, 0)

  def tblz(i, _):
    tblv[pl.ds(i * 16, 16)] = jnp.zeros((16,), jnp.int32)
    return 0
  lax.fori_loop(0, 16, tblz, 0)
  plsc.store_scatter(tblv, [io * 8], s0)
  plsc.store_scatter(tblv, [io * 8 + 1], p0)
  plsc.store_scatter(tblv, [io * 8 + 128], s1)
  plsc.store_scatter(tblv, [io * 8 + 129], p1)
  def tbl_dma(b, _):
    pltpu.sync_copy(tblv.at[pl.ds(b * 8, 8)],
                    tbl_o.at[pl.ds(b * NT * 8 + wid * 8, 8)])
    return 0
  lax.fori_loop(0, NB, tbl_dma, 0)


def _partition(dst, src, val):
  k = pl.kernel(
      _partition_body,
      out_type=(
          jax.ShapeDtypeStruct((EPAD,), jnp.int32),
          jax.ShapeDtypeStruct((EPAD,), jnp.int32),
          jax.ShapeDtypeStruct((EPAD,), jnp.float32),
          jax.ShapeDtypeStruct((NB * NT * 8,), jnp.int32),
      ),
      mesh=_mesh(),
      compiler_params=pltpu.CompilerParams(needs_layout_passes=False),
      scratch_types=(
          pltpu.VMEM((CH1,), jnp.int32),
          pltpu.VMEM((CH1,), jnp.int32),
          pltpu.VMEM((CH1,), jnp.float32),
          pltpu.VMEM((NB,), jnp.int32),
          pltpu.VMEM((NB,), jnp.int32),
          pltpu.VMEM((256,), jnp.int32),
          pltpu.VMEM((16,), jnp.int32),
          pltpu.VMEM((NB * STG,), jnp.int32),
          pltpu.VMEM((NB * STG,), jnp.int32),
          pltpu.VMEM((NB * STG,), jnp.float32),
          pltpu.VMEM((16,), jnp.int32),
          pltpu.VMEM((16,), jnp.float32),
          pltpu.SMEM((NB,), jnp.int32),
          pltpu.SemaphoreType.DMA,
      ),
  )
  return k(dst, src, val)


def _propagate_body(ego, srcs, dofs, vals, tbl, out,
                    tblsm, srcb0, dofb0, valb0, rows0,
                    srcb1, dofb1, valb1, rows1, accf,
                    semL0, semL1, semG0, semG1):
  wid = _wid()
  io = _io()
  pltpu.sync_copy(tbl.at[pl.ds(wid * NT * 8, NT * 8)],
                  tblsm.at[pl.ds(0, NT * 8)])

  zf = jnp.zeros((16,), jnp.float32)
  def zacc(i, _):
    for k in range(8):
      accf[pl.ds((i * 8 + k) * 16, 16)] = zf
    return 0
  lax.fori_loop(0, ACCW // 128, zacc, 0)

  slots = ((srcb0, dofb0, valb0, rows0, semL0, semG0),
           (srcb1, dofb1, valb1, rows1, semL1, semG1))

  def seg_body(st, _):
    tv = tblsm[pl.ds(st * 8, 16)]
    off = pl.multiple_of(tv[0], 16)
    cnt = tv[1]
    nch = lax.shift_right_logical(cnt + (KBC - 1), 8)

    def issue_loads(ci, s):
      sb, db, vb, _, sl, _ = slots[s]
      coff = off + ci * KBC
      pltpu.async_copy(srcs.at[pl.ds(coff, KBC)], sb, sl)
      pltpu.async_copy(dofs.at[pl.ds(coff, KBC)], db, sl)
      pltpu.async_copy(vals.at[pl.ds(coff, KBC)], vb, sl)

    def wait_loads(s):
      sb, db, vb, _, sl, _ = slots[s]
      pltpu.make_async_copy(srcs.at[pl.ds(0, KBC)], sb, sl).wait()
      pltpu.make_async_copy(dofs.at[pl.ds(0, KBC)], db, sl).wait()
      pltpu.make_async_copy(vals.at[pl.ds(0, KBC)], vb, sl).wait()

    def issue_gather(s):
      sb, _, _, rw, _, sg = slots[s]
      pltpu.async_copy(ego.at[sb], rw, sg)

    def wait_gather(s):
      sb, _, _, rw, _, sg = slots[s]
      pltpu.make_async_copy(ego.at[sb], rw, sg).wait()

    def compute(ci, s):
      _, db, vb, rw, _, _ = slots[s]
      ng = lax.shift_right_logical(jnp.minimum(KBC, cnt - ci * KBC), 4)
      @plsc.parallel_loop(0, ng)
      def _(g):
        e16 = g * 16 + io
        v = vb[pl.ds(g * 16, 16)]
        ao = db[pl.ds(g * 16, 16)]
        for c in range(EMB):
          cv = plsc.load_gather(rw, [e16, jnp.full((16,), c, jnp.int32)])
          plsc.addupdate_scatter(accf, [ao + c], cv * v)

    def chunk_step(ci, s):
      @pl.when(ci + 1 < nch)
      def _():
        wait_loads(1 - s)
        issue_gather(1 - s)
      wait_gather(s)
      compute(ci, s)
      @pl.when(ci + 2 < nch)
      def _():
        issue_loads(ci + 2, s)

    @pl.when(nch > 0)
    def _():
      issue_loads(0, 0)
      wait_loads(0)
      issue_gather(0)
      @pl.when(nch > 1)
      def _():
        issue_loads(1, 1)
      def pair_body(p, _):
        chunk_step(2 * p, 0)
        @pl.when(2 * p + 1 < nch)
        def _():
          chunk_step(2 * p + 1, 1)
        return 0
      lax.fori_loop(0, lax.shift_right_logical(nch + 1, 1), pair_body, 0)
    return 0
  lax.fori_loop(0, NT, seg_body, 0)

  def relu_body(i, _):
    for k in range(8):
      sl = pl.ds((i * 8 + k) * 16, 16)
      accf[sl] = jnp.maximum(accf[sl], 0.0)
    return 0
  lax.fori_loop(0, ACCW // 128, relu_body, 0)

  @pl.when(wid < NT - 1)
  def _():
    pltpu.sync_copy(accf, out.at[pl.ds(wid * ACCW, ACCW)])
  @pl.when(wid == NT - 1)
  def _():
    tailw = (N_NODES - (NT - 1) * RANGE) * EMB
    pltpu.sync_copy(accf.at[pl.ds(0, tailw)],
                    out.at[pl.ds((NT - 1) * ACCW, tailw)])


def _propagate(ego2d, srcs, dofs, vals, tbl):
  k = pl.kernel(
      _propagate_body,
      out_type=jax.ShapeDtypeStruct((N_NODES * EMB,), jnp.float32),
      mesh=_mesh(),
      compiler_params=pltpu.CompilerParams(
          needs_layout_passes=False, use_tc_tiling_on_sc=False),
      scratch_types=(
          pltpu.VMEM((NT * 8 + 16,), jnp.int32),
          pltpu.VMEM((KBC,), jnp.int32),
          pltpu.VMEM((KBC,), jnp.int32),
          pltpu.VMEM((KBC,), jnp.float32),
          pltpu.VMEM((KBC, EMB), jnp.float32),
          pltpu.VMEM((KBC,), jnp.int32),
          pltpu.VMEM((KBC,), jnp.int32),
          pltpu.VMEM((KBC,), jnp.float32),
          pltpu.VMEM((KBC, EMB), jnp.float32),
          pltpu.VMEM((ACCW,), jnp.float32),
          pltpu.SemaphoreType.DMA,
          pltpu.SemaphoreType.DMA,
          pltpu.SemaphoreType.DMA,
          pltpu.SemaphoreType.DMA,
      ),
  )
  return k(ego2d, srcs, dofs, vals, tbl)


def _mean_kernel(a, b, c, d, o):
  o[...] = 0.25 * (a[...] + b[...] + c[...] + d[...])


def _mean4(a, b, c, d):
  return pl.pallas_call(
      _mean_kernel,
      out_shape=jax.ShapeDtypeStruct((25000, 128), jnp.float32),
      grid=(25,),
      in_specs=[pl.BlockSpec((1000, 128), lambda i: (i, 0))] * 4,
      out_specs=pl.BlockSpec((1000, 128), lambda i: (i, 0)),
  )(a, b, c, d)


def kernel(user_emb, item_emb, adj_values, adj_indices):
  ego0 = jnp.concatenate([user_emb, item_emb], axis=0)
  dst = adj_indices[0]
  src = adj_indices[1]
  srcs, dofs, vals, tbl = _partition(dst, src, adj_values)
  egos = [ego0.reshape(-1)]
  e2d = ego0
  for _ in range(3):
    ef = _propagate(e2d, srcs, dofs, vals, tbl)
    egos.append(ef)
    e2d = ef.reshape(N_NODES, EMB)
  m = _mean4(*[x.reshape(25000, 128) for x in egos])
  m = m.reshape(N_NODES, EMB)
  return (m[:50000], m[50000:])


# speedup vs baseline: 5.7806x; 1.3682x over previous
import functools

import jax
import jax.numpy as jnp
from jax import lax
from jax.experimental import pallas as pl
from jax.experimental.pallas import tpu as pltpu
from jax.experimental.pallas import tpu_sc as plsc

N_NODES = 100000
EMB = 32
E = 1600000
NT = 32
PT = E // NT
NB = 32
RANGE = 3200
RSZ = PT + NB * 16
EPAD = NT * RSZ + 528
DUMP = EPAD - 16
CH1 = 2000
KBC = 256
ACCW = RANGE * EMB

_mesh = functools.partial(
    plsc.VectorSubcoreMesh, core_axis_name="c", subcore_axis_name="s")


def _wid():
  return lax.axis_index("s") * 2 + lax.axis_index("c")


def _bucket(d):
  q = lax.shift_right_logical(d, 7)
  return lax.shift_right_logical(q * 5243, 17)


def _io():
  return lax.iota(jnp.int32, 16)


def _ranks(sb, sbuf):
  io = _io()
  sbuf[...] = sb
  prev = plsc.load_gather(sbuf, [jnp.maximum(io - 1, 0)])
  nxt = plsc.load_gather(sbuf, [jnp.minimum(io + 1, 15)])
  first = jnp.logical_or(io == 0, sb != prev)
  is_end = jnp.logical_or(io == 15, sb != nxt)
  start = plsc.cummax(jnp.where(first, io, 0))
  rank = io - start
  return rank, is_end


STG = 160


def _partition_body(dst, src, val, srcs_o, dofs_o, vals_o, tbl_o,
                    dstb, srcb, valb, cntv,
                    hist, tblv, sbuf, stg_s, stg_d, stg_v,
                    zb16i, zb16f, gposS, sem):
  wid = _wid()
  slab = wid * PT
  regbase = wid * RSZ
  io = _io()

  hist[pl.ds(0, 16)] = jnp.zeros((16,), jnp.int32)
  hist[pl.ds(16, 16)] = jnp.zeros((16,), jnp.int32)
  zb16i[pl.ds(0, 16)] = jnp.zeros((16,), jnp.int32)
  zb16f[pl.ds(0, 16)] = jnp.zeros((16,), jnp.float32)

  def p1_chunk(c, _):
    pltpu.sync_copy(dst.at[pl.ds(slab + c * CH1, CH1)], dstb)
    def p1_vreg(i, _):
      d = dstb[pl.ds(i * 16, 16)]
      b = _bucket(d)
      sb, _ = plsc.sort_key_val(b, io)
      rank, is_end = _ranks(sb, sbuf)
      h = plsc.load_gather(hist, [sb])
      plsc.store_scatter(hist, [sb], h + rank + 1, mask=is_end)
      return 0
    lax.fori_loop(0, CH1 // 16, p1_vreg, 0)
    return 0
  lax.fori_loop(0, PT // CH1, p1_chunk, 0)

  h0 = hist[pl.ds(0, 16)]
  h1 = hist[pl.ds(16, 16)]
  p0 = jnp.bitwise_and(h0 + 15, -16)
  p1 = jnp.bitwise_and(h1 + 15, -16)
  c0 = plsc.cumsum(p0)
  c1 = plsc.cumsum(p1)
  tot0 = jnp.max(c0)
  s0 = regbase + (c0 - p0)
  s1 = regbase + (c1 - p1) + tot0

  for b in range(16):
    gposS[b] = s0[b]
    gposS[16 + b] = s1[b]

  cntv[pl.ds(0, 16)] = jnp.zeros((16,), jnp.int32)
  cntv[pl.ds(16, 16)] = jnp.zeros((16,), jnp.int32)

  def p2_chunk(c, _):
    coff = slab + c * CH1
    pltpu.sync_copy(dst.at[pl.ds(coff, CH1)], dstb)
    pltpu.sync_copy(src.at[pl.ds(coff, CH1)], srcb)
    pltpu.sync_copy(val.at[pl.ds(coff, CH1)], valb)
    def v_body(i, _):
      d = dstb[pl.ds(i * 16, 16)]
      b = _bucket(d)
      sb, lanes = plsc.sort_key_val(b, i * 16 + io)
      rank, is_end = _ranks(sb, sbuf)
      base = plsc.load_gather(cntv, [sb])
      pos = base + rank
      newc = pos + 1
      plsc.store_scatter(cntv, [sb], newc, mask=is_end)
      d_s = plsc.load_gather(dstb, [lanes])
      s_s = plsc.load_gather(srcb, [lanes])
      v_s = plsc.load_gather(valb, [lanes])
      doff_s = lax.shift_left(d_s - sb * RANGE, 5)
      addr = sb * STG + pos
      plsc.store_scatter(stg_s, [addr], s_s)
      plsc.store_scatter(stg_d, [addr], doff_s)
      plsc.store_scatter(stg_v, [addr], v_s)
      @pl.when(jnp.max(newc) >= 128)
      def _():
        cl = cntv[pl.ds(0, 16)]
        ch = cntv[pl.ds(16, 16)]
        for b2 in range(NB):
          cb = cl[b2] if b2 < 16 else ch[b2 - 16]
          sbase2 = b2 * STG
          @pl.when(cb >= 128)
          def _():
            g = pl.multiple_of(gposS[b2], 16)
            d1 = pltpu.async_copy(stg_s.at[pl.ds(sbase2, 128)],
                                  srcs_o.at[pl.ds(g, 128)], sem)
            d2 = pltpu.async_copy(stg_d.at[pl.ds(sbase2, 128)],
                                  dofs_o.at[pl.ds(g, 128)], sem)
            d3 = pltpu.async_copy(stg_v.at[pl.ds(sbase2, 128)],
                                  vals_o.at[pl.ds(g, 128)], sem)
            d1.wait()
            d2.wait()
            d3.wait()
            stg_s[pl.ds(sbase2, 16)] = stg_s[pl.ds(sbase2 + 128, 16)]
            stg_d[pl.ds(sbase2, 16)] = stg_d[pl.ds(sbase2 + 128, 16)]
            stg_v[pl.ds(sbase2, 16)] = stg_v[pl.ds(sbase2 + 128, 16)]
            gposS[b2] = g + 128
            plsc.store_scatter(cntv, [jnp.full((16,), b2, jnp.int32)],
                               jnp.full((16,), cb - 128, jnp.int32),
                               mask=io == 0)
      return 0
    lax.fori_loop(0, CH1 // 16, v_body, 0)
    return 0
  lax.fori_loop(0, PT // CH1, p2_chunk, 0)

  cl = cntv[pl.ds(0, 16)]
  ch = cntv[pl.ds(16, 16)]
  for b in range(NB):
    sbase = b * STG
    cnt = cl[b] if b < 16 else ch[b - 16]
    stg_s[pl.ds(sbase + cnt, 16)] = jnp.zeros((16,), jnp.int32)
    stg_d[pl.ds(sbase + cnt, 16)] = jnp.zeros((16,), jnp.int32)
    stg_v[pl.ds(sbase + cnt, 16)] = jnp.zeros((16,), jnp.float32)
    g0 = pl.multiple_of(gposS[b], 16)
    nfl = lax.shift_right_logical(cnt + 15, 4)
    def dr(i, _):
      o1 = pl.ds(sbase + i * 16, 16)
      o2 = pl.ds(g0 + i * 16, 16)
      d1 = pltpu.async_copy(stg_s.at[o1], srcs_o.at[o2], sem)
      d2 = pltpu.async_copy(stg_d.at[o1], dofs_o.at[o2], sem)
      d3 = pltpu.async_copy(stg_v.at[o1], vals_o.at[o2], sem)
      d1.wait()
      d2.wait()
      d3.wait()
      return 0
    lax.fori_loop(0, nfl, dr, 0)

  regend = regbase + jnp.max(c1) + tot0
  cap = jnp.where(wid == NT - 1, regbase + RSZ + 512, regbase + RSZ)
  nz = lax.shift_right_logical(cap - regend, 4)
  def z_body(i, _):
    o = pl.ds(pl.multiple_of(regend + i * 16, 16), 16)
    d1 = pltpu.async_copy(zb16i, srcs_o.at[o], sem)
    d2 = pltpu.async_copy(zb16i, dofs_o.at[o], sem)
    d3 = pltpu.async_copy(zb16f, vals_o.at[o], sem)
    d1.wait()
    d2.wait()
    d3.wait()
    return 0
  lax.fori_loop(0, nz, z_body, 0)

  def tblz(i, _):
    tblv[pl.ds(i * 16, 16)] = jnp.zeros((16,), jnp.int32)
    return 0
  lax.fori_loop(0, 16, tblz, 0)
  plsc.store_scatter(tblv, [io * 8], s0)
  plsc.store_scatter(tblv, [io * 8 + 1], p0)
  plsc.store_scatter(tblv, [io * 8 + 128], s1)
  plsc.store_scatter(tblv, [io * 8 + 129], p1)
  def tbl_dma(b, _):
    pltpu.sync_copy(tblv.at[pl.ds(b * 8, 8)],
                    tbl_o.at[pl.ds(b * NT * 8 + wid * 8, 8)])
    return 0
  lax.fori_loop(0, NB, tbl_dma, 0)


def _partition(dst, src, val):
  k = pl.kernel(
      _partition_body,
      out_type=(
          jax.ShapeDtypeStruct((EPAD,), jnp.int32),
          jax.ShapeDtypeStruct((EPAD,), jnp.int32),
          jax.ShapeDtypeStruct((EPAD,), jnp.float32),
          jax.ShapeDtypeStruct((NB * NT * 8,), jnp.int32),
      ),
      mesh=_mesh(),
      compiler_params=pltpu.CompilerParams(needs_layout_passes=False),
      scratch_types=(
          pltpu.VMEM((CH1,), jnp.int32),
          pltpu.VMEM((CH1,), jnp.int32),
          pltpu.VMEM((CH1,), jnp.float32),
          pltpu.VMEM((NB,), jnp.int32),
          pltpu.VMEM((NB,), jnp.int32),
          pltpu.VMEM((256,), jnp.int32),
          pltpu.VMEM((16,), jnp.int32),
          pltpu.VMEM((NB * STG,), jnp.int32),
          pltpu.VMEM((NB * STG,), jnp.int32),
          pltpu.VMEM((NB * STG,), jnp.float32),
          pltpu.VMEM((16,), jnp.int32),
          pltpu.VMEM((16,), jnp.float32),
          pltpu.SMEM((NB,), jnp.int32),
          pltpu.SemaphoreType.DMA,
      ),
  )
  return k(dst, src, val)


def _propagate_body(ego_pk, srcs, dofs, vals, tbl, out, outpk,
                    tblsm, srcb0, dofb0, valb0, rows0,
                    srcb1, dofb1, valb1, rows1, accf,
                    semL0, semL1, semG0, semG1):
  wid = _wid()
  io = _io()
  sid = lax.axis_index("s")
  pltpu.sync_copy(tbl.at[pl.ds(wid * NT * 8, NT * 8)],
                  tblsm.at[pl.ds(0, NT * 8)])

  del sid
  zf = jnp.zeros((16,), jnp.float32)
  def zacc(i, _):
    for k in range(8):
      accf[pl.ds((i * 8 + k) * 16, 16)] = zf
    return 0
  lax.fori_loop(0, ACCW // 128, zacc, 0)

  slots = ((srcb0, dofb0, valb0, rows0, semL0, semG0),
           (srcb1, dofb1, valb1, rows1, semL1, semG1))

  def seg_body(st, _):
    tv = tblsm[pl.ds(st * 8, 16)]
    off = pl.multiple_of(tv[0], 16)
    cnt = tv[1]
    nch = lax.shift_right_logical(cnt + (KBC - 1), 8)

    def issue_loads(ci, s):
      sb, db, vb, _, sl, _ = slots[s]
      coff = off + ci * KBC
      pltpu.async_copy(srcs.at[pl.ds(coff, KBC)], sb, sl)
      pltpu.async_copy(dofs.at[pl.ds(coff, KBC)], db, sl)
      pltpu.async_copy(vals.at[pl.ds(coff, KBC)], vb, sl)

    def wait_loads(s):
      sb, db, vb, _, sl, _ = slots[s]
      pltpu.make_async_copy(srcs.at[pl.ds(0, KBC)], sb, sl).wait()
      pltpu.make_async_copy(dofs.at[pl.ds(0, KBC)], db, sl).wait()
      pltpu.make_async_copy(vals.at[pl.ds(0, KBC)], vb, sl).wait()

    def issue_gather(s):
      sb, _, _, rw, _, sg = slots[s]
      pltpu.async_copy(ego_pk.at[sb], rw, sg)

    def wait_gather(s):
      sb, _, _, rw, _, sg = slots[s]
      pltpu.make_async_copy(ego_pk.at[sb], rw, sg).wait()

    def compute(ci, s):
      _, db, vb, rw, _, _ = slots[s]
      ng = lax.shift_right_logical(jnp.minimum(KBC, cnt - ci * KBC), 4)
      @plsc.parallel_loop(0, ng)
      def _(g):
        e16 = g * 16 + io
        v = vb[pl.ds(g * 16, 16)]
        ao = db[pl.ds(g * 16, 16)]
        for w in range(EMB // 2):
          pk = plsc.load_gather(rw, [e16, jnp.full((16,), w, jnp.int32)])
          bf = plsc.bitcast(pk, jnp.bfloat16)
          a, b = plsc.unpack(bf, format=plsc.PackFormat.INTERLEAVED)
          plsc.addupdate_scatter(accf, [ao + w], a * v)
          plsc.addupdate_scatter(accf, [ao + (w + 16)], b * v)

    def chunk_step(ci, s):
      @pl.when(ci + 1 < nch)
      def _():
        wait_loads(1 - s)
        issue_gather(1 - s)
      wait_gather(s)
      compute(ci, s)
      @pl.when(ci + 2 < nch)
      def _():
        issue_loads(ci + 2, s)

    @pl.when(nch > 0)
    def _():
      issue_loads(0, 0)
      wait_loads(0)
      issue_gather(0)
      @pl.when(nch > 1)
      def _():
        issue_loads(1, 1)
      def pair_body(p, _):
        chunk_step(2 * p, 0)
        @pl.when(2 * p + 1 < nch)
        def _():
          chunk_step(2 * p + 1, 1)
        return 0
      lax.fori_loop(0, lax.shift_right_logical(nch + 1, 1), pair_body, 0)
    return 0
  lax.fori_loop(0, NT, seg_body, 0)

  def relu_body(i, _):
    for k in range(8):
      sl = pl.ds((i * 8 + k) * 16, 16)
      accf[sl] = jnp.maximum(accf[sl], 0.0)
    return 0
  lax.fori_loop(0, ACCW // 128, relu_body, 0)

  @pl.when(wid < NT - 1)
  def _():
    pltpu.sync_copy(accf, out.at[pl.ds(wid * ACCW, ACCW)])
  @pl.when(wid == NT - 1)
  def _():
    tailw = (N_NODES - (NT - 1) * RANGE) * EMB
    pltpu.sync_copy(accf.at[pl.ds(0, tailw)],
                    out.at[pl.ds((NT - 1) * ACCW, tailw)])

  def pk_body(i, _):
    for k in range(4):
      r = i * 4 + k
      a = accf[pl.ds(r * 32, 16)]
      b = accf[pl.ds(r * 32 + 16, 16)]
      pw = plsc.bitcast(
          plsc.pack(a, b, format=plsc.PackFormat.INTERLEAVED), jnp.float32)
      accf[pl.ds(r * 16, 16)] = pw
    return 0
  lax.fori_loop(0, RANGE // 4, pk_body, 0)
  pkw = RANGE * (EMB // 2)
  @pl.when(wid < NT - 1)
  def _():
    pltpu.sync_copy(accf.at[pl.ds(0, pkw)], outpk.at[pl.ds(wid * pkw, pkw)])
  @pl.when(wid == NT - 1)
  def _():
    tailp = (N_NODES - (NT - 1) * RANGE) * (EMB // 2)
    pltpu.sync_copy(accf.at[pl.ds(0, tailp)],
                    outpk.at[pl.ds((NT - 1) * pkw, tailp)])


def _propagate(ego_pk, srcs, dofs, vals, tbl):
  k = pl.kernel(
      _propagate_body,
      out_type=(
          jax.ShapeDtypeStruct((N_NODES * EMB,), jnp.float32),
          jax.ShapeDtypeStruct((N_NODES * EMB // 2,), jnp.float32),
      ),
      mesh=_mesh(),
      compiler_params=pltpu.CompilerParams(
          needs_layout_passes=False, use_tc_tiling_on_sc=False),
      scratch_types=(
          pltpu.VMEM((NT * 8 + 16,), jnp.int32),
          pltpu.VMEM((KBC,), jnp.int32),
          pltpu.VMEM((KBC,), jnp.int32),
          pltpu.VMEM((KBC,), jnp.float32),
          pltpu.VMEM((KBC, EMB // 2), jnp.float32),
          pltpu.VMEM((KBC,), jnp.int32),
          pltpu.VMEM((KBC,), jnp.int32),
          pltpu.VMEM((KBC,), jnp.float32),
          pltpu.VMEM((KBC, EMB // 2), jnp.float32),
          pltpu.VMEM((ACCW,), jnp.float32),
          pltpu.SemaphoreType.DMA,
          pltpu.SemaphoreType.DMA,
          pltpu.SemaphoreType.DMA,
          pltpu.SemaphoreType.DMA,
      ),
  )
  return k(ego_pk, srcs, dofs, vals, tbl)


def _mean_kernel(a, b, c, d, o):
  o[...] = 0.25 * (a[...] + b[...] + c[...] + d[...])


def _mean4(a, b, c, d):
  return pl.pallas_call(
      _mean_kernel,
      out_shape=jax.ShapeDtypeStruct((25000, 128), jnp.float32),
      grid=(25,),
      in_specs=[pl.BlockSpec((1000, 128), lambda i: (i, 0))] * 4,
      out_specs=pl.BlockSpec((1000, 128), lambda i: (i, 0)),
  )(a, b, c, d)


def _pack_host(x2d):
  ab = x2d.astype(jnp.bfloat16)
  st = jnp.stack([ab[:, :EMB // 2], ab[:, EMB // 2:]], axis=-1)
  return jax.lax.bitcast_convert_type(st, jnp.float32)


def kernel(user_emb, item_emb, adj_values, adj_indices):
  ego0 = jnp.concatenate([user_emb, item_emb], axis=0)
  dst = adj_indices[0]
  src = adj_indices[1]
  srcs, dofs, vals, tbl = _partition(dst, src, adj_values)
  egos = [ego0.reshape(-1)]
  e_pk = _pack_host(ego0)
  for _ in range(3):
    ef, epk = _propagate(e_pk, srcs, dofs, vals, tbl)
    egos.append(ef)
    e_pk = epk.reshape(N_NODES, EMB // 2)
  m = _mean4(*[x.reshape(25000, 128) for x in egos])
  m = m.reshape(N_NODES, EMB)
  return (m[:50000], m[50000:])


# speedup vs baseline: 5.9608x; 1.0312x over previous
import functools

import jax
import jax.numpy as jnp
from jax import lax
from jax.experimental import pallas as pl
from jax.experimental.pallas import tpu as pltpu
from jax.experimental.pallas import tpu_sc as plsc

N_NODES = 100000
EMB = 32
E = 1600000
NT = 32
PT = E // NT
NB = 32
RANGE = 3200
RSZ = PT + NB * 16
EPAD = NT * RSZ + 528
DUMP = EPAD - 16
CH1 = 2000
KBC = 512
KBCS = 9
ACCW = RANGE * EMB

_mesh = functools.partial(
    plsc.VectorSubcoreMesh, core_axis_name="c", subcore_axis_name="s")


def _wid():
  return lax.axis_index("s") * 2 + lax.axis_index("c")


def _bucket(d):
  q = lax.shift_right_logical(d, 7)
  return lax.shift_right_logical(q * 5243, 17)


def _io():
  return lax.iota(jnp.int32, 16)


def _ranks(sb, sbuf):
  io = _io()
  sbuf[...] = sb
  prev = plsc.load_gather(sbuf, [jnp.maximum(io - 1, 0)])
  nxt = plsc.load_gather(sbuf, [jnp.minimum(io + 1, 15)])
  first = jnp.logical_or(io == 0, sb != prev)
  is_end = jnp.logical_or(io == 15, sb != nxt)
  start = plsc.cummax(jnp.where(first, io, 0))
  rank = io - start
  return rank, is_end


STG = 160


def _partition_body(dst, src, val, srcs_o, dofs_o, vals_o, tbl_o,
                    dstb, srcb, valb, cntv,
                    hist, tblv, sbuf, stg_s, stg_d, stg_v,
                    zb16i, zb16f, gposS, sem):
  wid = _wid()
  slab = wid * PT
  regbase = wid * RSZ
  io = _io()

  hist[pl.ds(0, 16)] = jnp.zeros((16,), jnp.int32)
  hist[pl.ds(16, 16)] = jnp.zeros((16,), jnp.int32)
  zb16i[pl.ds(0, 16)] = jnp.zeros((16,), jnp.int32)
  zb16f[pl.ds(0, 16)] = jnp.zeros((16,), jnp.float32)

  def p1_chunk(c, _):
    pltpu.sync_copy(dst.at[pl.ds(slab + c * CH1, CH1)], dstb)
    def p1_vreg(i, _):
      d = dstb[pl.ds(i * 16, 16)]
      b = _bucket(d)
      sb, _ = plsc.sort_key_val(b, io)
      rank, is_end = _ranks(sb, sbuf)
      h = plsc.load_gather(hist, [sb])
      plsc.store_scatter(hist, [sb], h + rank + 1, mask=is_end)
      return 0
    lax.fori_loop(0, CH1 // 16, p1_vreg, 0)
    return 0
  lax.fori_loop(0, PT // CH1, p1_chunk, 0)

  h0 = hist[pl.ds(0, 16)]
  h1 = hist[pl.ds(16, 16)]
  p0 = jnp.bitwise_and(h0 + 15, -16)
  p1 = jnp.bitwise_and(h1 + 15, -16)
  c0 = plsc.cumsum(p0)
  c1 = plsc.cumsum(p1)
  tot0 = jnp.max(c0)
  s0 = regbase + (c0 - p0)
  s1 = regbase + (c1 - p1) + tot0

  for b in range(16):
    gposS[b] = s0[b]
    gposS[16 + b] = s1[b]

  cntv[pl.ds(0, 16)] = jnp.zeros((16,), jnp.int32)
  cntv[pl.ds(16, 16)] = jnp.zeros((16,), jnp.int32)

  def p2_chunk(c, _):
    coff = slab + c * CH1
    pltpu.sync_copy(dst.at[pl.ds(coff, CH1)], dstb)
    pltpu.sync_copy(src.at[pl.ds(coff, CH1)], srcb)
    pltpu.sync_copy(val.at[pl.ds(coff, CH1)], valb)
    def v_body(i, _):
      d = dstb[pl.ds(i * 16, 16)]
      b = _bucket(d)
      sb, lanes = plsc.sort_key_val(b, i * 16 + io)
      rank, is_end = _ranks(sb, sbuf)
      base = plsc.load_gather(cntv, [sb])
      pos = base + rank
      newc = pos + 1
      plsc.store_scatter(cntv, [sb], newc, mask=is_end)
      d_s = plsc.load_gather(dstb, [lanes])
      s_s = plsc.load_gather(srcb, [lanes])
      v_s = plsc.load_gather(valb, [lanes])
      doff_s = lax.shift_left(d_s - sb * RANGE, 5)
      addr = sb * STG + pos
      plsc.store_scatter(stg_s, [addr], s_s)
      plsc.store_scatter(stg_d, [addr], doff_s)
      plsc.store_scatter(stg_v, [addr], v_s)
      @pl.when(jnp.max(newc) >= 128)
      def _():
        cl = cntv[pl.ds(0, 16)]
        ch = cntv[pl.ds(16, 16)]
        for b2 in range(NB):
          cb = cl[b2] if b2 < 16 else ch[b2 - 16]
          sbase2 = b2 * STG
          @pl.when(cb >= 128)
          def _():
            g = pl.multiple_of(gposS[b2], 16)
            d1 = pltpu.async_copy(stg_s.at[pl.ds(sbase2, 128)],
                                  srcs_o.at[pl.ds(g, 128)], sem)
            d2 = pltpu.async_copy(stg_d.at[pl.ds(sbase2, 128)],
                                  dofs_o.at[pl.ds(g, 128)], sem)
            d3 = pltpu.async_copy(stg_v.at[pl.ds(sbase2, 128)],
                                  vals_o.at[pl.ds(g, 128)], sem)
            d1.wait()
            d2.wait()
            d3.wait()
            stg_s[pl.ds(sbase2, 16)] = stg_s[pl.ds(sbase2 + 128, 16)]
            stg_d[pl.ds(sbase2, 16)] = stg_d[pl.ds(sbase2 + 128, 16)]
            stg_v[pl.ds(sbase2, 16)] = stg_v[pl.ds(sbase2 + 128, 16)]
            gposS[b2] = g + 128
            plsc.store_scatter(cntv, [jnp.full((16,), b2, jnp.int32)],
                               jnp.full((16,), cb - 128, jnp.int32),
                               mask=io == 0)
      return 0
    lax.fori_loop(0, CH1 // 16, v_body, 0)
    return 0
  lax.fori_loop(0, PT // CH1, p2_chunk, 0)

  cl = cntv[pl.ds(0, 16)]
  ch = cntv[pl.ds(16, 16)]
  for b in range(NB):
    sbase = b * STG
    cnt = cl[b] if b < 16 else ch[b - 16]
    stg_s[pl.ds(sbase + cnt, 16)] = jnp.zeros((16,), jnp.int32)
    stg_d[pl.ds(sbase + cnt, 16)] = jnp.zeros((16,), jnp.int32)
    stg_v[pl.ds(sbase + cnt, 16)] = jnp.zeros((16,), jnp.float32)
    g0 = pl.multiple_of(gposS[b], 16)
    nfl = lax.shift_right_logical(cnt + 15, 4)
    def dr(i, _):
      o1 = pl.ds(sbase + i * 16, 16)
      o2 = pl.ds(g0 + i * 16, 16)
      d1 = pltpu.async_copy(stg_s.at[o1], srcs_o.at[o2], sem)
      d2 = pltpu.async_copy(stg_d.at[o1], dofs_o.at[o2], sem)
      d3 = pltpu.async_copy(stg_v.at[o1], vals_o.at[o2], sem)
      d1.wait()
      d2.wait()
      d3.wait()
      return 0
    lax.fori_loop(0, nfl, dr, 0)

  regend = regbase + jnp.max(c1) + tot0
  cap = jnp.where(wid == NT - 1, regbase + RSZ + 512, regbase + RSZ)
  nz = lax.shift_right_logical(cap - regend, 4)
  def z_body(i, _):
    o = pl.ds(pl.multiple_of(regend + i * 16, 16), 16)
    d1 = pltpu.async_copy(zb16i, srcs_o.at[o], sem)
    d2 = pltpu.async_copy(zb16i, dofs_o.at[o], sem)
    d3 = pltpu.async_copy(zb16f, vals_o.at[o], sem)
    d1.wait()
    d2.wait()
    d3.wait()
    return 0
  lax.fori_loop(0, nz, z_body, 0)

  def tblz(i, _):
    tblv[pl.ds(i * 16, 16)] = jnp.zeros((16,), jnp.int32)
    return 0
  lax.fori_loop(0, 16, tblz, 0)
  plsc.store_scatter(tblv, [io * 8], s0)
  plsc.store_scatter(tblv, [io * 8 + 1], p0)
  plsc.store_scatter(tblv, [io * 8 + 128], s1)
  plsc.store_scatter(tblv, [io * 8 + 129], p1)
  def tbl_dma(b, _):
    pltpu.sync_copy(tblv.at[pl.ds(b * 8, 8)],
                    tbl_o.at[pl.ds(b * NT * 8 + wid * 8, 8)])
    return 0
  lax.fori_loop(0, NB, tbl_dma, 0)


def _partition(dst, src, val):
  k = pl.kernel(
      _partition_body,
      out_type=(
          jax.ShapeDtypeStruct((EPAD,), jnp.int32),
          jax.ShapeDtypeStruct((EPAD,), jnp.int32),
          jax.ShapeDtypeStruct((EPAD,), jnp.float32),
          jax.ShapeDtypeStruct((NB * NT * 8,), jnp.int32),
      ),
      mesh=_mesh(),
      compiler_params=pltpu.CompilerParams(needs_layout_passes=False),
      scratch_types=(
          pltpu.VMEM((CH1,), jnp.int32),
          pltpu.VMEM((CH1,), jnp.int32),
          pltpu.VMEM((CH1,), jnp.float32),
          pltpu.VMEM((NB,), jnp.int32),
          pltpu.VMEM((NB,), jnp.int32),
          pltpu.VMEM((256,), jnp.int32),
          pltpu.VMEM((16,), jnp.int32),
          pltpu.VMEM((NB * STG,), jnp.int32),
          pltpu.VMEM((NB * STG,), jnp.int32),
          pltpu.VMEM((NB * STG,), jnp.float32),
          pltpu.VMEM((16,), jnp.int32),
          pltpu.VMEM((16,), jnp.float32),
          pltpu.SMEM((NB,), jnp.int32),
          pltpu.SemaphoreType.DMA,
      ),
  )
  return k(dst, src, val)


def _propagate_body(ego_pk, srcs, dofs, vals, tbl, out, outpk,
                    tblsm, srcb0, dofb0, valb0, rows0,
                    srcb1, dofb1, valb1, rows1, accf,
                    semL0, semL1, semG0, semG1):
  wid = _wid()
  io = _io()
  sid = lax.axis_index("s")
  pltpu.sync_copy(tbl.at[pl.ds(wid * NT * 8, NT * 8)],
                  tblsm.at[pl.ds(0, NT * 8)])

  del sid
  zf = jnp.zeros((16,), jnp.float32)
  def zacc(i, _):
    for k in range(8):
      accf[pl.ds((i * 8 + k) * 16, 16)] = zf
    return 0
  lax.fori_loop(0, ACCW // 128, zacc, 0)

  slots = ((srcb0, dofb0, valb0, rows0, semL0, semG0),
           (srcb1, dofb1, valb1, rows1, semL1, semG1))

  def seg_body(st, _):
    tv = tblsm[pl.ds(st * 8, 16)]
    off = pl.multiple_of(tv[0], 16)
    cnt = tv[1]
    nch = lax.shift_right_logical(cnt + (KBC - 1), KBCS)

    def issue_loads(ci, s):
      sb, db, vb, _, sl, _ = slots[s]
      coff = off + ci * KBC
      pltpu.async_copy(srcs.at[pl.ds(coff, KBC)], sb, sl)
      pltpu.async_copy(dofs.at[pl.ds(coff, KBC)], db, sl)
      pltpu.async_copy(vals.at[pl.ds(coff, KBC)], vb, sl)

    def wait_loads(s):
      sb, db, vb, _, sl, _ = slots[s]
      pltpu.make_async_copy(srcs.at[pl.ds(0, KBC)], sb, sl).wait()
      pltpu.make_async_copy(dofs.at[pl.ds(0, KBC)], db, sl).wait()
      pltpu.make_async_copy(vals.at[pl.ds(0, KBC)], vb, sl).wait()

    def issue_gather(s):
      sb, _, _, rw, _, sg = slots[s]
      pltpu.async_copy(ego_pk.at[sb], rw, sg)

    def wait_gather(s):
      sb, _, _, rw, _, sg = slots[s]
      pltpu.make_async_copy(ego_pk.at[sb], rw, sg).wait()

    def compute(ci, s):
      _, db, vb, rw, _, _ = slots[s]
      ng = lax.shift_right_logical(jnp.minimum(KBC, cnt - ci * KBC), 4)
      @plsc.parallel_loop(0, ng)
      def _(g):
        e16 = g * 16 + io
        v = vb[pl.ds(g * 16, 16)]
        ao = db[pl.ds(g * 16, 16)]
        for w in range(EMB // 2):
          pk = plsc.load_gather(rw, [e16, jnp.full((16,), w, jnp.int32)])
          bf = plsc.bitcast(pk, jnp.bfloat16)
          a, b = plsc.unpack(bf, format=plsc.PackFormat.INTERLEAVED)
          plsc.addupdate_scatter(accf, [ao + w], a * v)
          plsc.addupdate_scatter(accf, [ao + (w + 16)], b * v)

    def chunk_step(ci, s):
      @pl.when(ci + 1 < nch)
      def _():
        wait_loads(1 - s)
        issue_gather(1 - s)
      wait_gather(s)
      compute(ci, s)
      @pl.when(ci + 2 < nch)
      def _():
        issue_loads(ci + 2, s)

    @pl.when(nch > 0)
    def _():
      issue_loads(0, 0)
      wait_loads(0)
      issue_gather(0)
      @pl.when(nch > 1)
      def _():
        issue_loads(1, 1)
      def pair_body(p, _):
        chunk_step(2 * p, 0)
        @pl.when(2 * p + 1 < nch)
        def _():
          chunk_step(2 * p + 1, 1)
        return 0
      lax.fori_loop(0, lax.shift_right_logical(nch + 1, 1), pair_body, 0)
    return 0
  lax.fori_loop(0, NT, seg_body, 0)

  def relu_body(i, _):
    for k in range(8):
      sl = pl.ds((i * 8 + k) * 16, 16)
      accf[sl] = jnp.maximum(accf[sl], 0.0)
    return 0
  lax.fori_loop(0, ACCW // 128, relu_body, 0)

  @pl.when(wid < NT - 1)
  def _():
    pltpu.sync_copy(accf, out.at[pl.ds(wid * ACCW, ACCW)])
  @pl.when(wid == NT - 1)
  def _():
    tailw = (N_NODES - (NT - 1) * RANGE) * EMB
    pltpu.sync_copy(accf.at[pl.ds(0, tailw)],
                    out.at[pl.ds((NT - 1) * ACCW, tailw)])

  def pk_body(i, _):
    for k in range(4):
      r = i * 4 + k
      a = accf[pl.ds(r * 32, 16)]
      b = accf[pl.ds(r * 32 + 16, 16)]
      pw = plsc.bitcast(
          plsc.pack(a, b, format=plsc.PackFormat.INTERLEAVED), jnp.float32)
      accf[pl.ds(r * 16, 16)] = pw
    return 0
  lax.fori_loop(0, RANGE // 4, pk_body, 0)
  pkw = RANGE * (EMB // 2)
  @pl.when(wid < NT - 1)
  def _():
    pltpu.sync_copy(accf.at[pl.ds(0, pkw)], outpk.at[pl.ds(wid * pkw, pkw)])
  @pl.when(wid == NT - 1)
  def _():
    tailp = (N_NODES - (NT - 1) * RANGE) * (EMB // 2)
    pltpu.sync_copy(accf.at[pl.ds(0, tailp)],
                    outpk.at[pl.ds((NT - 1) * pkw, tailp)])


def _propagate(ego_pk, srcs, dofs, vals, tbl):
  k = pl.kernel(
      _propagate_body,
      out_type=(
          jax.ShapeDtypeStruct((N_NODES * EMB,), jnp.float32),
          jax.ShapeDtypeStruct((N_NODES * EMB // 2,), jnp.float32),
      ),
      mesh=_mesh(),
      compiler_params=pltpu.CompilerParams(
          needs_layout_passes=False, use_tc_tiling_on_sc=False),
      scratch_types=(
          pltpu.VMEM((NT * 8 + 16,), jnp.int32),
          pltpu.VMEM((KBC,), jnp.int32),
          pltpu.VMEM((KBC,), jnp.int32),
          pltpu.VMEM((KBC,), jnp.float32),
          pltpu.VMEM((KBC, EMB // 2), jnp.float32),
          pltpu.VMEM((KBC,), jnp.int32),
          pltpu.VMEM((KBC,), jnp.int32),
          pltpu.VMEM((KBC,), jnp.float32),
          pltpu.VMEM((KBC, EMB // 2), jnp.float32),
          pltpu.VMEM((ACCW,), jnp.float32),
          pltpu.SemaphoreType.DMA,
          pltpu.SemaphoreType.DMA,
          pltpu.SemaphoreType.DMA,
          pltpu.SemaphoreType.DMA,
      ),
  )
  return k(ego_pk, srcs, dofs, vals, tbl)


def _mean_kernel(a, b, c, d, o):
  o[...] = 0.25 * (a[...] + b[...] + c[...] + d[...])


def _mean4(a, b, c, d):
  return pl.pallas_call(
      _mean_kernel,
      out_shape=jax.ShapeDtypeStruct((25000, 128), jnp.float32),
      grid=(25,),
      in_specs=[pl.BlockSpec((1000, 128), lambda i: (i, 0))] * 4,
      out_specs=pl.BlockSpec((1000, 128), lambda i: (i, 0)),
  )(a, b, c, d)


def _pack_host(x2d):
  ab = x2d.astype(jnp.bfloat16)
  st = jnp.stack([ab[:, :EMB // 2], ab[:, EMB // 2:]], axis=-1)
  return jax.lax.bitcast_convert_type(st, jnp.float32)


def kernel(user_emb, item_emb, adj_values, adj_indices):
  ego0 = jnp.concatenate([user_emb, item_emb], axis=0)
  dst = adj_indices[0]
  src = adj_indices[1]
  srcs, dofs, vals, tbl = _partition(dst, src, adj_values)
  egos = [ego0.reshape(-1)]
  e_pk = _pack_host(ego0)
  for _ in range(3):
    ef, epk = _propagate(e_pk, srcs, dofs, vals, tbl)
    egos.append(ef)
    e_pk = epk.reshape(N_NODES, EMB // 2)
  m = _mean4(*[x.reshape(25000, 128) for x in egos])
  m = m.reshape(N_NODES, EMB)
  return (m[:50000], m[50000:])
